# baseline (speedup 1.0000x reference)
import functools

import jax
import jax.numpy as jnp
from jax import lax
from jax.experimental import pallas as pl
from jax.experimental.pallas import tpu as pltpu
from jax.experimental.pallas import tpu_sc as plsc

_VOCAB = 100000
_MAX_POS = 2048
_D = 768
_BATCH = 4
_SEQ = 2048
_N = _BATCH * _SEQ
_NC = 2
_NS = 16
_NW = _NC * _NS
_PER_W = _N // _NW
_CHUNK = 64
_NCHUNK = _PER_W // _CHUNK
_LANES = 16
_VECS_PER_ROW = _D // _LANES


def _emb_body(ids_hbm, tok_hbm, pos_hbm, out_hbm, idx_v, tok_v, pos_v, sem):
    wid = lax.axis_index("s") * _NC + lax.axis_index("c")
    base = wid * _PER_W
    pos_base = lax.rem(base, _SEQ)

    pltpu.sync_copy(ids_hbm.at[pl.ds(base, _PER_W)], idx_v)

    for c in range(_NCHUNK):
        off = c * _CHUNK
        pltpu.async_copy(tok_hbm.at[idx_v.at[pl.ds(off, _CHUNK)]], tok_v,
                         sem).wait()
        pltpu.sync_copy(pos_hbm.at[pl.ds(pos_base + off, _CHUNK)], pos_v)

        def _row(r, carry):
            def _col(j, carry2):
                s = pl.ds(j * _LANES, _LANES)
                tok_v[r, s] = tok_v[r, s] + pos_v[r, s]
                return carry2
            return lax.fori_loop(0, _VECS_PER_ROW, _col, carry)
        lax.fori_loop(0, _CHUNK, _row, 0)

        pltpu.sync_copy(tok_v, out_hbm.at[pl.ds(base + off, _CHUNK)])


@jax.jit
def _emb(ids_flat, token_table, pos_table):
    mesh = plsc.VectorSubcoreMesh(core_axis_name="c", subcore_axis_name="s")
    run = functools.partial(
        pl.kernel,
        mesh=mesh,
        out_type=jax.ShapeDtypeStruct((_N, _D), jnp.float32),
        scratch_types=[
            pltpu.VMEM((_PER_W,), jnp.int32),
            pltpu.VMEM((_CHUNK, _D), jnp.float32),
            pltpu.VMEM((_CHUNK, _D), jnp.float32),
            pltpu.SemaphoreType.DMA,
        ],
    )(_emb_body)
    return run(ids_flat, token_table, pos_table)


def kernel(input_ids, token_table, pos_table):
    ids_flat = input_ids.reshape(_N).astype(jnp.int32)
    out = _emb(ids_flat, token_table, pos_table)
    return out.reshape(_BATCH, _SEQ, _D)
